# Initial kernel scaffold; baseline (speedup 1.0000x reference)
import jax
import jax.numpy as jnp
from jax.experimental import pallas as pl


def kernel(x, e, edge_index, W1, b1, g1, beta1, W2, b2):
    raise NotImplementedError("write your pallas kernel here")


# speedup vs baseline: 3.8281x; 3.8281x over previous
import functools

import jax
import jax.numpy as jnp
from jax import lax
from jax.experimental import pallas as pl
from jax.experimental.pallas import tpu as pltpu
from jax.experimental.pallas import tpu_sc as plsc

N_NODES = 10000
D = 128
N_EDGES = 320000

NC = 2
NS = 16
NW = NC * NS
CHUNK = 128
CHUNKS_PER_TILE = 79
EDGES_PER_TILE = CHUNK * CHUNKS_PER_TILE
E_PAD = EDGES_PER_TILE * NW
N_PAD = 10240
ROWS_PER_TILE = N_PAD // NS


def _make_sc_messages():
    mesh = plsc.VectorSubcoreMesh(core_axis_name="c", subcore_axis_name="s")

    @functools.partial(
        pl.kernel,
        mesh=mesh,
        out_type=jax.ShapeDtypeStruct((NC * N_PAD, D), jnp.float32),
        scratch_types=[
            pltpu.VMEM((CHUNK,), jnp.int32),
            pltpu.VMEM((CHUNK,), jnp.int32),
            pltpu.VMEM((CHUNK,), jnp.float32),
            pltpu.VMEM((CHUNK, D), jnp.float32),
            pltpu.VMEM((CHUNK, D), jnp.float32),
            pltpu.VMEM_SHARED((N_PAD, D), jnp.float32),
            pltpu.SemaphoreType.DMA,
            pltpu.SemaphoreType.DMA,
        ],
    )
    def body(x_hbm, src_hbm, dst_hbm, e_hbm, out_hbm,
             idx_s, idx_d, ev, rows_s, rows_d, acc, sem1, sem2):
        cid = lax.axis_index("c")
        sid = lax.axis_index("s")
        wid = cid * NS + sid

        zero = jnp.zeros((16,), jnp.float32)

        def zrow(i, carry):
            for r in range(D // 16):
                rows_s[i, pl.ds(r * 16, 16)] = zero
            return carry

        lax.fori_loop(0, CHUNK, zrow, 0)
        r0 = sid * ROWS_PER_TILE
        for t in range(ROWS_PER_TILE // CHUNK):
            pltpu.sync_copy(rows_s,
                            acc.at[pl.ds(r0 + t * CHUNK, CHUNK)])
        plsc.subcore_barrier()

        base0 = wid * EDGES_PER_TILE

        def chunk_body(c, carry):
            base = base0 + c * CHUNK
            pltpu.sync_copy(src_hbm.at[pl.ds(base, CHUNK)], idx_s)
            pltpu.sync_copy(dst_hbm.at[pl.ds(base, CHUNK)], idx_d)
            pltpu.sync_copy(e_hbm.at[pl.ds(base, CHUNK)], ev)
            cp1 = pltpu.async_copy(x_hbm.at[idx_s], rows_s, sem1)
            cp2 = pltpu.async_copy(x_hbm.at[idx_d], rows_d, sem2)
            cp1.wait()
            cp2.wait()

            def scale(g, inner):
                ev16 = ev[pl.ds(g * 16, 16)]
                i0 = g * 16
                for j in range(16):
                    eb = jnp.full((16,), ev16[j], jnp.float32)
                    for r in range(D // 16):
                        sl = pl.ds(r * 16, 16)
                        rows_s[i0 + j, sl] = rows_s[i0 + j, sl] * eb
                        rows_d[i0 + j, sl] = rows_d[i0 + j, sl] * eb
                return inner

            lax.fori_loop(0, CHUNK // 16, scale, 0)
            pltpu.sync_copy(rows_s, acc.at[idx_d], add=True)
            pltpu.sync_copy(rows_d, acc.at[idx_s], add=True)
            return carry

        lax.fori_loop(0, CHUNKS_PER_TILE, chunk_body, 0)

        plsc.subcore_barrier()
        out_base = cid * N_PAD + r0
        pltpu.sync_copy(acc.at[pl.ds(r0, ROWS_PER_TILE)],
                        out_hbm.at[pl.ds(out_base, ROWS_PER_TILE)])

    return body


_SC_CACHE = []


def _sc_messages():
    if not _SC_CACHE:
        _SC_CACHE.append(_make_sc_messages())
    return _SC_CACHE[0]

_R = 1000


def _mlp_body(mi_ref, x_ref, w1a_ref, w1b_ref, vecs_ref, w2_ref, out_ref):
    mi = mi_ref[0] + mi_ref[1]
    h = (
        jnp.dot(mi, w1a_ref[...], preferred_element_type=jnp.float32,
                precision=lax.Precision.HIGHEST)
        + jnp.dot(x_ref[...], w1b_ref[...], preferred_element_type=jnp.float32,
                  precision=lax.Precision.HIGHEST)
        + vecs_ref[0:1, :]
    )
    mean = jnp.mean(h, axis=1, keepdims=True)
    var = jnp.mean((h - mean) ** 2, axis=1, keepdims=True)
    h = (h - mean) * lax.rsqrt(var + 1e-5) * vecs_ref[1:2, :] + vecs_ref[2:3, :]
    h = jnp.tanh(h)
    out_ref[...] = (
        jnp.dot(h, w2_ref[...], preferred_element_type=jnp.float32,
                precision=lax.Precision.HIGHEST)
        + vecs_ref[3:4, :]
    )


def _mlp(mi2, x, w1a, w1b, vecs, w2):
    grid = (N_NODES // _R,)
    return pl.pallas_call(
        _mlp_body,
        grid=grid,
        in_specs=[
            pl.BlockSpec((2, _R, D), lambda i: (0, i, 0)),
            pl.BlockSpec((_R, D), lambda i: (i, 0)),
            pl.BlockSpec((D, D), lambda i: (0, 0)),
            pl.BlockSpec((D, D), lambda i: (0, 0)),
            pl.BlockSpec((8, D), lambda i: (0, 0)),
            pl.BlockSpec((D, D), lambda i: (0, 0)),
        ],
        out_specs=pl.BlockSpec((_R, D), lambda i: (i, 0)),
        out_shape=jax.ShapeDtypeStruct((N_NODES, D), jnp.float32),
    )(mi2, x, w1a, w1b, vecs, w2)


def kernel(x, e, edge_index, W1, b1, g1, beta1, W2, b2):
    src = edge_index[0].astype(jnp.int32)
    dst = edge_index[1].astype(jnp.int32)
    pad = E_PAD - N_EDGES
    src = jnp.pad(src, (0, pad))
    dst = jnp.pad(dst, (0, pad))
    ep = jnp.pad(e, (0, pad))
    partials = _sc_messages()(x, src, dst, ep)
    mi2 = partials.reshape(2, N_PAD, D)
    vecs = (
        jnp.zeros((8, D), jnp.float32)
        .at[0].set(b1).at[1].set(g1).at[2].set(beta1).at[3].set(b2)
    )
    return _mlp(mi2, x, W1[:D], W1[D:], vecs, W2)
